# baseline (speedup 1.0000x reference)
import functools

import jax
import jax.numpy as jnp
from jax import lax
from jax.experimental import pallas as pl
from jax.experimental.pallas import tpu as pltpu
from jax.experimental.pallas import tpu_sc as plsc

LANE = 128
K = 1
CHUNK = K * LANE
PARTS = 8


@functools.lru_cache(maxsize=None)
def _build_gather(num_idx: int, vocab: int):
    info = plsc.get_sparse_core_info()
    nw = info.num_cores * info.num_subcores
    per_w = num_idx // nw
    assert num_idx % nw == 0 and per_w % CHUNK == 0
    nchunk = per_w // CHUNK
    nmain = nchunk // 2 * 2

    mesh = plsc.VectorSubcoreMesh(core_axis_name="c", subcore_axis_name="s")

    @functools.partial(
        pl.kernel,
        out_type=jax.ShapeDtypeStruct((num_idx, LANE), jnp.float32),
        mesh=mesh,
        scratch_types=[
            pltpu.VMEM((per_w,), jnp.int32),
            pltpu.VMEM((CHUNK, LANE), jnp.float32),
            pltpu.VMEM((CHUNK, LANE), jnp.float32),
            pltpu.SemaphoreType.DMA,
            pltpu.SemaphoreType.DMA,
            pltpu.SemaphoreType.DMA,
            pltpu.SemaphoreType.DMA,
        ],
    )
    def emb(idx_hbm, table_hbm, out_hbm, idxb, rows0, rows1,
            g0, g1, o0, o1):
        wid = lax.axis_index("s") * info.num_cores + lax.axis_index("c")
        base = wid * per_w
        row_bufs = (rows0, rows1)
        gsems = (g0, g1)
        osems = (o0, o1)

        pltpu.sync_copy(idx_hbm.at[pl.ds(base, per_w)], idxb)

        def chunk(c, b):
            rows_b = row_bufs[b]
            row0 = pl.multiple_of(base + c * CHUNK, CHUNK)

            @pl.when(c >= 2)
            def _():
                pltpu.make_async_copy(
                    rows_b, out_hbm.at[pl.ds(row0, CHUNK)], osems[b]
                ).wait()

            copies = [
                pltpu.async_copy(
                    table_hbm.at[idxb.at[pl.ds(c * CHUNK + j * LANE,
                                               LANE)]],
                    rows_b.at[pl.ds(j * LANE, LANE)],
                    gsems[b],
                )
                for j in range(K)
            ]
            for cp in copies:
                cp.wait()

            pltpu.async_copy(
                rows_b, out_hbm.at[pl.ds(row0, CHUNK)], osems[b])

        @pl.loop(0, nmain, step=2)
        def _(c0):
            for b in range(2):
                chunk(c0 + b, b)

        for c in range(nmain, nchunk):
            chunk(c, c % 2)

        for b in range(2):
            pltpu.make_async_copy(
                row_bufs[b], out_hbm.at[pl.ds(b * CHUNK, CHUNK)], osems[b]
            ).wait()

    return emb


@functools.lru_cache(maxsize=None)
def _build_pad(vocab: int, dim: int, rows: int = 8192):
    def body(t_ref, o_ref):
        o_ref[:, :dim] = jnp.swapaxes(t_ref[...], 0, 1)
        o_ref[:, dim:] = jnp.zeros((rows, LANE - dim), jnp.float32)

    return pl.pallas_call(
        body,
        grid=(pl.cdiv(vocab, rows),),
        in_specs=[pl.BlockSpec((dim, rows), lambda i: (0, i))],
        out_specs=pl.BlockSpec((rows, LANE), lambda i: (i, 0)),
        out_shape=jax.ShapeDtypeStruct((vocab, LANE), jnp.float32),
    )


@functools.lru_cache(maxsize=None)
def _build_narrow(num_idx: int, part: int, n_parts: int, dim: int,
                  rows: int = 4096):
    part_rows = num_idx // n_parts
    assert part_rows % rows == 0
    nblk = part_rows // rows
    g0 = part * nblk

    def body(p_ref, *rest):
        o_ref = rest[-1]
        o_ref[...] = jnp.swapaxes(p_ref[...], 0, 1)[:dim, :]

    in_specs = [pl.BlockSpec((rows, LANE), lambda i: (i, 0))]
    kwargs = {}
    if part > 0:
        in_specs.append(pl.BlockSpec(memory_space=pl.ANY))
        kwargs["input_output_aliases"] = {1: 0}

    return pl.pallas_call(
        body,
        grid=(nblk,),
        in_specs=in_specs,
        out_specs=pl.BlockSpec((dim, rows), lambda i: (0, i + g0)),
        out_shape=jax.ShapeDtypeStruct((dim, num_idx), jnp.float32),
        **kwargs,
    )


def kernel(x, table):
    bsz, hist = x.shape
    vocab, dim = table.shape
    num_idx = bsz * hist
    part_rows = num_idx // PARTS
    idx = jnp.swapaxes(x, 0, 1).reshape(num_idx).astype(jnp.int32)
    table_pad = _build_pad(vocab, dim)(jnp.swapaxes(table, 0, 1))
    gather = _build_gather(part_rows, vocab)
    out = None
    for p in range(PARTS):
        rows_pad = gather(
            lax.slice(idx, (p * part_rows,), ((p + 1) * part_rows,)),
            table_pad)
        narrow = _build_narrow(num_idx, p, PARTS, dim)
        out = narrow(rows_pad) if p == 0 else narrow(rows_pad, out)
    return jnp.transpose(out.reshape(dim, hist, bsz), (2, 1, 0))


# speedup vs baseline: 1.4684x; 1.4684x over previous
import functools

import jax
import jax.numpy as jnp
from jax import lax
from jax.experimental import pallas as pl
from jax.experimental.pallas import tpu as pltpu
from jax.experimental.pallas import tpu_sc as plsc

LANE = 128
K = 1
CHUNK = K * LANE
PARTS = 8


@functools.lru_cache(maxsize=None)
def _build_gather(num_idx: int, vocab: int):
    info = plsc.get_sparse_core_info()
    nw = info.num_cores * info.num_subcores
    per_w = num_idx // nw
    assert num_idx % nw == 0 and per_w % CHUNK == 0
    nchunk = per_w // CHUNK
    nmain = nchunk // 2 * 2

    mesh = plsc.VectorSubcoreMesh(core_axis_name="c", subcore_axis_name="s")

    @functools.partial(
        pl.kernel,
        out_type=jax.ShapeDtypeStruct((num_idx, LANE), jnp.float32),
        mesh=mesh,
        scratch_types=[
            pltpu.VMEM((per_w,), jnp.int32),
            pltpu.VMEM((CHUNK, LANE), jnp.float32),
            pltpu.VMEM((CHUNK, LANE), jnp.float32),
            pltpu.SemaphoreType.DMA,
            pltpu.SemaphoreType.DMA,
            pltpu.SemaphoreType.DMA,
            pltpu.SemaphoreType.DMA,
        ],
    )
    def emb(idx_hbm, table_hbm, out_hbm, idxb, rows0, rows1,
            g0, g1, o0, o1):
        wid = lax.axis_index("s") * info.num_cores + lax.axis_index("c")
        base = wid * per_w
        row_bufs = (rows0, rows1)
        gsems = (g0, g1)
        osems = (o0, o1)

        pltpu.sync_copy(idx_hbm.at[pl.ds(base, per_w)], idxb)

        def chunk(c, b):
            rows_b = row_bufs[b]
            row0 = pl.multiple_of(base + c * CHUNK, CHUNK)

            @pl.when(c >= 2)
            def _():
                pltpu.make_async_copy(
                    rows_b, out_hbm.at[pl.ds(row0, CHUNK)], osems[b]
                ).wait()

            copies = [
                pltpu.async_copy(
                    table_hbm.at[idxb.at[pl.ds(c * CHUNK + j * LANE,
                                               LANE)]],
                    rows_b.at[pl.ds(j * LANE, LANE)],
                    gsems[b],
                )
                for j in range(K)
            ]
            for cp in copies:
                cp.wait()

            pltpu.async_copy(
                rows_b, out_hbm.at[pl.ds(row0, CHUNK)], osems[b])

        @pl.loop(0, nmain, step=2)
        def _(c0):
            for b in range(2):
                chunk(c0 + b, b)

        for c in range(nmain, nchunk):
            chunk(c, c % 2)

        for b in range(2):
            pltpu.make_async_copy(
                row_bufs[b], out_hbm.at[pl.ds(b * CHUNK, CHUNK)], osems[b]
            ).wait()

    return emb


@functools.lru_cache(maxsize=None)
def _build_pad(vocab: int, dim: int, rows: int = 8192):
    def body(t_ref, o_ref):
        o_ref[:, :dim] = jnp.swapaxes(t_ref[...], 0, 1)
        o_ref[:, dim:] = jnp.zeros((rows, LANE - dim), jnp.float32)

    return pl.pallas_call(
        body,
        grid=(pl.cdiv(vocab, rows),),
        in_specs=[pl.BlockSpec((dim, rows), lambda i: (0, i))],
        out_specs=pl.BlockSpec((rows, LANE), lambda i: (i, 0)),
        out_shape=jax.ShapeDtypeStruct((vocab, LANE), jnp.float32),
    )


@functools.lru_cache(maxsize=None)
def _build_narrow(bsz: int, hist: int, part: int, n_parts: int, dim: int,
                  hblk: int = 8):
    part_b = bsz // n_parts
    assert hist % hblk == 0
    nblk = hist // hblk
    b0 = part * part_b

    def body(p_ref, *rest):
        o_ref = rest[-1]
        for h in range(hblk):
            o_ref[:, h, :] = jnp.swapaxes(p_ref[:, h, :dim], 0, 1)

    in_specs = [pl.BlockSpec((part_b, hblk, LANE), lambda i: (0, i, 0))]
    kwargs = {}
    if part > 0:
        in_specs.append(pl.BlockSpec(memory_space=pl.ANY))
        kwargs["input_output_aliases"] = {1: 0}

    return pl.pallas_call(
        body,
        grid=(nblk,),
        in_specs=in_specs,
        out_specs=pl.BlockSpec((dim, hblk, part_b),
                               lambda i: (0, i, b0 // part_b)),
        out_shape=jax.ShapeDtypeStruct((dim, hist, bsz), jnp.float32),
        **kwargs,
    )


def kernel(x, table):
    bsz, hist = x.shape
    vocab, dim = table.shape
    num_idx = bsz * hist
    part_rows = num_idx // PARTS
    part_b = bsz // PARTS
    idx = x.reshape(num_idx).astype(jnp.int32)
    table_pad = _build_pad(vocab, dim)(jnp.swapaxes(table, 0, 1))
    gather = _build_gather(part_rows, vocab)
    out = None
    for p in range(PARTS):
        rows_pad = gather(
            lax.slice(idx, (p * part_rows,), ((p + 1) * part_rows,)),
            table_pad)
        rows_pad = rows_pad.reshape(part_b, hist, LANE)
        narrow = _build_narrow(bsz, hist, p, PARTS, dim)
        out = narrow(rows_pad) if p == 0 else narrow(rows_pad, out)
    return jnp.transpose(out, (2, 1, 0))


# speedup vs baseline: 1.4835x; 1.0102x over previous
import functools

import jax
import jax.numpy as jnp
from jax import lax
from jax.experimental import pallas as pl
from jax.experimental.pallas import tpu as pltpu
from jax.experimental.pallas import tpu_sc as plsc

LANE = 128
K = 1
CHUNK = K * LANE
PARTS = 8


@functools.lru_cache(maxsize=None)
def _build_gather(num_idx: int, vocab: int):
    info = plsc.get_sparse_core_info()
    nw = info.num_cores * info.num_subcores
    per_w = num_idx // nw
    assert num_idx % nw == 0 and per_w % CHUNK == 0
    nchunk = per_w // CHUNK
    nmain = nchunk // 2 * 2

    mesh = plsc.VectorSubcoreMesh(core_axis_name="c", subcore_axis_name="s")

    @functools.partial(
        pl.kernel,
        out_type=jax.ShapeDtypeStruct((num_idx, LANE), jnp.float32),
        mesh=mesh,
        scratch_types=[
            pltpu.VMEM((per_w,), jnp.int32),
            pltpu.VMEM((CHUNK, LANE), jnp.float32),
            pltpu.VMEM((CHUNK, LANE), jnp.float32),
            pltpu.SemaphoreType.DMA,
            pltpu.SemaphoreType.DMA,
            pltpu.SemaphoreType.DMA,
            pltpu.SemaphoreType.DMA,
        ],
    )
    def emb(idx_hbm, table_hbm, out_hbm, idxb, rows0, rows1,
            g0, g1, o0, o1):
        wid = lax.axis_index("s") * info.num_cores + lax.axis_index("c")
        base = wid * per_w
        row_bufs = (rows0, rows1)
        gsems = (g0, g1)
        osems = (o0, o1)

        pltpu.sync_copy(idx_hbm.at[pl.ds(base, per_w)], idxb)

        def chunk(c, b):
            rows_b = row_bufs[b]
            row0 = pl.multiple_of(base + c * CHUNK, CHUNK)

            @pl.when(c >= 2)
            def _():
                pltpu.make_async_copy(
                    rows_b, out_hbm.at[pl.ds(row0, CHUNK)], osems[b]
                ).wait()

            copies = [
                pltpu.async_copy(
                    table_hbm.at[idxb.at[pl.ds(c * CHUNK + j * LANE,
                                               LANE)]],
                    rows_b.at[pl.ds(j * LANE, LANE)],
                    gsems[b],
                )
                for j in range(K)
            ]
            for cp in copies:
                cp.wait()

            pltpu.async_copy(
                rows_b, out_hbm.at[pl.ds(row0, CHUNK)], osems[b])

        @pl.loop(0, nmain, step=2)
        def _(c0):
            for b in range(2):
                chunk(c0 + b, b)

        for c in range(nmain, nchunk):
            chunk(c, c % 2)

        for b in range(2):
            pltpu.make_async_copy(
                row_bufs[b], out_hbm.at[pl.ds(b * CHUNK, CHUNK)], osems[b]
            ).wait()

    return emb


@functools.lru_cache(maxsize=None)
def _build_pad(vocab: int, dim: int, rows: int = 16384):
    def body(t_ref, o_ref):
        o_ref[:, :dim] = jnp.swapaxes(t_ref[...], 0, 1)
        o_ref[:, dim:] = jnp.zeros((rows, LANE - dim), jnp.float32)

    return pl.pallas_call(
        body,
        grid=(pl.cdiv(vocab, rows),),
        in_specs=[pl.BlockSpec((dim, rows), lambda i: (0, i))],
        out_specs=pl.BlockSpec((rows, LANE), lambda i: (i, 0)),
        out_shape=jax.ShapeDtypeStruct((vocab, LANE), jnp.float32),
    )


@functools.lru_cache(maxsize=None)
def _build_narrow(bsz: int, hist: int, part: int, n_parts: int, dim: int,
                  hblk: int = 8):
    part_b = bsz // n_parts
    assert hist % hblk == 0
    nblk = hist // hblk
    b0 = part * part_b

    def body(p_ref, *rest):
        o_ref = rest[-1]
        for h in range(hblk):
            o_ref[:, h, :] = jnp.swapaxes(p_ref[:, h, :dim], 0, 1)

    in_specs = [pl.BlockSpec((part_b, hblk, LANE), lambda i: (0, i, 0))]
    kwargs = {}
    if part > 0:
        in_specs.append(pl.BlockSpec(memory_space=pl.ANY))
        kwargs["input_output_aliases"] = {1: 0}

    return pl.pallas_call(
        body,
        grid=(nblk,),
        in_specs=in_specs,
        out_specs=pl.BlockSpec((dim, hblk, part_b),
                               lambda i: (0, i, b0 // part_b)),
        out_shape=jax.ShapeDtypeStruct((dim, hist, bsz), jnp.float32),
        **kwargs,
    )


def kernel(x, table):
    bsz, hist = x.shape
    vocab, dim = table.shape
    num_idx = bsz * hist
    part_rows = num_idx // PARTS
    part_b = bsz // PARTS
    idx = x.reshape(num_idx).astype(jnp.int32)
    table_pad = _build_pad(vocab, dim)(jnp.swapaxes(table, 0, 1))
    gather = _build_gather(part_rows, vocab)
    out = None
    for p in range(PARTS):
        rows_pad = gather(
            lax.slice(idx, (p * part_rows,), ((p + 1) * part_rows,)),
            table_pad)
        rows_pad = rows_pad.reshape(part_b, hist, LANE)
        narrow = _build_narrow(bsz, hist, p, PARTS, dim)
        out = narrow(rows_pad) if p == 0 else narrow(rows_pad, out)
    return jnp.transpose(out, (2, 1, 0))
